# baseline (speedup 1.0000x reference)
import dataclasses
import functools

import jax
import jax.numpy as jnp
from jax import lax
from jax.experimental import pallas as pl
from jax.experimental.pallas import tpu as pltpu
from jax.experimental.pallas import tpu_sc as plsc

_NN = 16
_BALL2 = 0.2
_B, _N, _C = 4, 4096, 3
_RB = 512
_BN = _B * _N
_NW = 32
_QW = _BN // _NW
_QBLK = 16
_NMOM = 19


def _beta_max(r):
    beta = jnp.full_like(r, 2.0)
    for _ in range(24):
        f = beta * beta * beta - 3.0 * beta - 2.0 * r
        fp = 3.0 * beta * beta - 3.0
        beta = jnp.clip(beta - f / (fp + 1e-12), 1.0, 2.0)
    return beta


def _cond_from_gram(axx, ayy, azz, axy, axz, ayz):
    q = (axx + ayy + azz) * (1.0 / 3.0)
    p1 = axy * axy + axz * axz + ayz * ayz
    dxx = axx - q
    dyy = ayy - q
    dzz = azz - q
    p2 = dxx * dxx + dyy * dyy + dzz * dzz + 2.0 * p1
    p = jnp.sqrt(jnp.maximum(p2 * (1.0 / 6.0), 0.0))
    pinv = jnp.where(p > 1e-30, 1.0 / jnp.maximum(p, 1e-30), 0.0)
    bxx = dxx * pinv
    byy = dyy * pinv
    bzz = dzz * pinv
    bxy = axy * pinv
    bxz = axz * pinv
    byz = ayz * pinv
    detb = (bxx * (byy * bzz - byz * byz)
            - bxy * (bxy * bzz - byz * bxz)
            + bxz * (bxy * byz - byy * bxz))
    r = jnp.clip(0.5 * detb, -1.0, 1.0)
    lmax = q + p * _beta_max(r)
    lmin = q - p * _beta_max(-r)
    s0 = jnp.sqrt(jnp.maximum(lmax, 0.0))
    s2 = jnp.sqrt(jnp.maximum(lmin, 0.0))
    return s0 / (s0 + s2 + 1e-30)


def _knn_body(refc_ref, reft_ref, idx_ref, msk_ref):
    i = pl.program_id(1)
    tx = refc_ref[:, 0:1]
    ty = refc_ref[:, 1:2]
    tz = refc_ref[:, 2:3]
    ax = reft_ref[0:1, :]
    ay = reft_ref[1:2, :]
    az = reft_ref[2:3, :]

    dx = tx - ax
    dy = ty - ay
    dz = tz - az
    d = dx * dx + dy * dy + dz * dz

    ji = lax.broadcasted_iota(jnp.int32, (_N, _RB), 0)
    work = (lax.bitcast_convert_type(d, jnp.int32) & jnp.int32(-8192)) | (
        jnp.where(d < _BALL2, ji, ji | jnp.int32(4096)))
    row_g = i * _RB + lax.broadcasted_iota(jnp.int32, (1, _RB), 1)
    idx_rows = [row_g]
    msk_rows = [jnp.ones((1, _RB), jnp.float32)]
    prev = row_g
    big = jnp.int32(2147483647)
    for _ in range(_NN - 1):
        v = jnp.min(jnp.where(work > prev, work, big), axis=0, keepdims=True)
        idx_rows.append(v & jnp.int32(4095))
        msk_rows.append(((v & jnp.int32(4096)) == 0).astype(jnp.float32))
        prev = v
    idx_ref[...] = jnp.concatenate(idx_rows, axis=0)
    msk_ref[...] = jnp.concatenate(msk_rows, axis=0)


def _knn_call(ref_points, ref_t):
    nsteps = _N // _RB
    steps_per_w = _QW // _RB

    def omap(b, i):
        s = b * nsteps + i
        return (s // steps_per_w, 0, s % steps_per_w)

    return pl.pallas_call(
        _knn_body,
        grid=(_B, nsteps),
        in_specs=[
            pl.BlockSpec((None, _N, _C), lambda b, i: (b, 0, 0)),
            pl.BlockSpec((None, _C, _RB), lambda b, i: (b, 0, i)),
        ],
        out_specs=[
            pl.BlockSpec((None, _NN, _RB), omap),
            pl.BlockSpec((None, _NN, _RB), omap),
        ],
        out_shape=[
            jax.ShapeDtypeStruct((_NW, _NN, _QW), jnp.int32),
            jax.ShapeDtypeStruct((_NW, _NN, _QW), jnp.float32),
        ],
    )(ref_points, ref_t)


def _moments_sc(coords, idx16, msk16):
    mesh = plsc.VectorSubcoreMesh(core_axis_name="c", subcore_axis_name="s")

    cp = pltpu.CompilerParams()
    if "needs_layout_passes" in pltpu.CompilerParams.__dataclass_fields__:
        cp = dataclasses.replace(cp, needs_layout_passes=False)

    @functools.partial(
        pl.kernel,
        mesh=mesh,
        compiler_params=cp,
        out_type=jax.ShapeDtypeStruct((_NW, _NMOM, _QW), jnp.float32),
        scratch_types=[
            pltpu.VMEM((_N,), jnp.float32),
            pltpu.VMEM((_N,), jnp.float32),
            pltpu.VMEM((_N,), jnp.float32),
            pltpu.VMEM((_N,), jnp.float32),
            pltpu.VMEM((_N,), jnp.float32),
            pltpu.VMEM((_N,), jnp.float32),
            pltpu.VMEM((_NN, _QW), jnp.int32),
            pltpu.VMEM((_NN, _QW), jnp.float32),
            pltpu.VMEM((_NMOM, _QW), jnp.float32),
            pltpu.SemaphoreType.DMA,
        ],
    )
    def sc_kernel(coords_hbm, idx_hbm, msk_hbm, out_hbm,
                  rxv, ryv, rzv, pxv, pyv, pzv, iv, mv, ov, sem):
        wid = lax.axis_index("s") * 2 + lax.axis_index("c")
        qbase = wid * _QW
        batch = qbase // _N
        pltpu.async_copy(coords_hbm.at[batch, 0], rxv, sem).wait()
        pltpu.async_copy(coords_hbm.at[batch, 1], ryv, sem).wait()
        pltpu.async_copy(coords_hbm.at[batch, 2], rzv, sem).wait()
        pltpu.async_copy(coords_hbm.at[batch, 3], pxv, sem).wait()
        pltpu.async_copy(coords_hbm.at[batch, 4], pyv, sem).wait()
        pltpu.async_copy(coords_hbm.at[batch, 5], pzv, sem).wait()
        pltpu.async_copy(idx_hbm.at[wid], iv, sem).wait()
        pltpu.async_copy(msk_hbm.at[wid], mv, sem).wait()

        qoff = qbase - batch * _N

        @pl.loop(0, _QW, step=_QBLK)
        def _(base):
            qx = rxv[pl.ds(qoff + base, _QBLK)]
            qy = ryv[pl.ds(qoff + base, _QBLK)]
            qz = rzv[pl.ds(qoff + base, _QBLK)]
            zero = jnp.zeros((_QBLK,), jnp.float32)
            acc = [zero] * _NMOM
            for k in range(_NN):
                nidx = iv[k, pl.ds(base, _QBLK)]
                mk = mv[k, pl.ds(base, _QBLK)]
                gx = plsc.load_gather(rxv, [nidx]) - qx
                gy = plsc.load_gather(ryv, [nidx]) - qy
                gz = plsc.load_gather(rzv, [nidx]) - qz
                hx = plsc.load_gather(pxv, [nidx])
                hy = plsc.load_gather(pyv, [nidx])
                hz = plsc.load_gather(pzv, [nidx])
                acc[0] = acc[0] + mk
                acc[1] = acc[1] + mk * gx
                acc[2] = acc[2] + mk * gy
                acc[3] = acc[3] + mk * gz
                acc[4] = acc[4] + mk * gx * gx
                acc[5] = acc[5] + mk * gy * gy
                acc[6] = acc[6] + mk * gz * gz
                acc[7] = acc[7] + mk * gx * gy
                acc[8] = acc[8] + mk * gx * gz
                acc[9] = acc[9] + mk * gy * gz
                acc[10] = acc[10] + hx
                acc[11] = acc[11] + hy
                acc[12] = acc[12] + hz
                acc[13] = acc[13] + hx * hx
                acc[14] = acc[14] + hy * hy
                acc[15] = acc[15] + hz * hz
                acc[16] = acc[16] + hx * hy
                acc[17] = acc[17] + hx * hz
                acc[18] = acc[18] + hy * hz
            for s in range(_NMOM):
                ov[s, pl.ds(base, _QBLK)] = acc[s]

        pltpu.async_copy(ov, out_hbm.at[wid], sem).wait()

    return sc_kernel(coords, idx16, msk16)


def _finish_body(mom_ref, refq_ref, out_ref):
    g = pl.program_id(0)

    nb = mom_ref[0:1, :]
    m1x = mom_ref[1:2, :]
    m1y = mom_ref[2:3, :]
    m1z = mom_ref[3:4, :]
    m2xx = mom_ref[4:5, :]
    m2yy = mom_ref[5:6, :]
    m2zz = mom_ref[6:7, :]
    m2xy = mom_ref[7:8, :]
    m2xz = mom_ref[8:9, :]
    m2yz = mom_ref[9:10, :]
    p1x = mom_ref[10:11, :]
    p1y = mom_ref[11:12, :]
    p1z = mom_ref[12:13, :]
    p2xx = mom_ref[13:14, :]
    p2yy = mom_ref[14:15, :]
    p2zz = mom_ref[15:16, :]
    p2xy = mom_ref[16:17, :]
    p2xz = mom_ref[17:18, :]
    p2yz = mom_ref[18:19, :]
    ax = refq_ref[0:1, :]
    ay = refq_ref[1:2, :]
    az = refq_ref[2:3, :]

    nbinv = 1.0 / nb
    csx = m1x * nbinv
    csy = m1y * nbinv
    csz = m1z * nbinv
    cx = csx + ax
    cy = csy + ay
    cz = csz + az
    w = _NN - nb
    gxx = m2xx - nb * csx * csx + w * cx * cx
    gyy = m2yy - nb * csy * csy + w * cy * cy
    gzz = m2zz - nb * csz * csz + w * cz * cz
    gxy = m2xy - nb * csx * csy + w * cx * cy
    gxz = m2xz - nb * csx * csz + w * cx * cz
    gyz = m2yz - nb * csy * csz + w * cy * cz
    cond_ref = _cond_from_gram(gxx, gyy, gzz, gxy, gxz, gyz)

    cpx = p1x * nbinv
    cpy = p1y * nbinv
    cpz = p1z * nbinv
    hxx = p2xx - 2.0 * cpx * p1x + _NN * cpx * cpx
    hyy = p2yy - 2.0 * cpy * p1y + _NN * cpy * cpy
    hzz = p2zz - 2.0 * cpz * p1z + _NN * cpz * cpz
    hxy = p2xy - cpx * p1y - cpy * p1x + _NN * cpx * cpy
    hxz = p2xz - cpx * p1z - cpz * p1x + _NN * cpx * cpz
    hyz = p2yz - cpy * p1z - cpz * p1y + _NN * cpy * cpz
    cond_p = _cond_from_gram(hxx, hyy, hzz, hxy, hxz, hyz)

    diff = cond_p - cond_ref
    partial = jnp.sum(diff * diff, axis=1, keepdims=True)

    @pl.when(g == 0)
    def _():
        out_ref[0:1, 0:1] = jnp.zeros((1, 1), jnp.float32)

    out_ref[0:1, 0:1] = out_ref[0:1, 0:1] + partial

    @pl.when(g == _NW - 1)
    def _():
        out_ref[0:1, 0:1] = out_ref[0:1, 0:1] * (1.0 / _BN)


def _finish_call(mom, refq):
    return pl.pallas_call(
        _finish_body,
        grid=(_NW,),
        in_specs=[
            pl.BlockSpec((None, _NMOM, _QW), lambda g: (g, 0, 0)),
            pl.BlockSpec((_C, _QW), lambda g: (0, g)),
        ],
        out_specs=pl.BlockSpec((1, 1), lambda g: (0, 0)),
        out_shape=jax.ShapeDtypeStruct((1, 1), jnp.float32),
    )(mom, refq)


def kernel(ref_points, points):
    ref_t = ref_points.transpose(0, 2, 1)
    pts_t = points.transpose(0, 2, 1)
    coords = jnp.concatenate([ref_t, pts_t], axis=1)
    idx16, msk16 = _knn_call(ref_points, ref_t)
    mom = _moments_sc(coords, idx16, msk16)
    refq = ref_t.transpose(1, 0, 2).reshape(_C, _BN)
    out = _finish_call(mom, refq)
    return out[0, 0]


# speedup vs baseline: 1.0152x; 1.0152x over previous
import dataclasses
import functools

import jax
import jax.numpy as jnp
from jax import lax
from jax.experimental import pallas as pl
from jax.experimental.pallas import tpu as pltpu
from jax.experimental.pallas import tpu_sc as plsc

_NN = 16
_BALL2 = 0.2
_B, _N, _C = 4, 4096, 3
_RB = 512
_BN = _B * _N
_NW = 32
_QW = _BN // _NW
_QBLK = 16
_NMOM = 19


def _beta_max(r):
    beta = jnp.full_like(r, 2.0)
    for _ in range(24):
        f = beta * beta * beta - 3.0 * beta - 2.0 * r
        fp = 3.0 * beta * beta - 3.0
        beta = jnp.clip(beta - f / (fp + 1e-12), 1.0, 2.0)
    return beta


def _cond_from_gram(axx, ayy, azz, axy, axz, ayz):
    q = (axx + ayy + azz) * (1.0 / 3.0)
    p1 = axy * axy + axz * axz + ayz * ayz
    dxx = axx - q
    dyy = ayy - q
    dzz = azz - q
    p2 = dxx * dxx + dyy * dyy + dzz * dzz + 2.0 * p1
    p = jnp.sqrt(jnp.maximum(p2 * (1.0 / 6.0), 0.0))
    pinv = jnp.where(p > 1e-30, 1.0 / jnp.maximum(p, 1e-30), 0.0)
    bxx = dxx * pinv
    byy = dyy * pinv
    bzz = dzz * pinv
    bxy = axy * pinv
    bxz = axz * pinv
    byz = ayz * pinv
    detb = (bxx * (byy * bzz - byz * byz)
            - bxy * (bxy * bzz - byz * bxz)
            + bxz * (bxy * byz - byy * bxz))
    r = jnp.clip(0.5 * detb, -1.0, 1.0)
    lmax = q + p * _beta_max(r)
    lmin = q - p * _beta_max(-r)
    s0 = jnp.sqrt(jnp.maximum(lmax, 0.0))
    s2 = jnp.sqrt(jnp.maximum(lmin, 0.0))
    return s0 / (s0 + s2 + 1e-30)


def _knn_body(refc_ref, reft_ref, idx_ref, msk_ref):
    i = pl.program_id(1)
    tx = refc_ref[:, 0:1]
    ty = refc_ref[:, 1:2]
    tz = refc_ref[:, 2:3]
    ax = reft_ref[0:1, :]
    ay = reft_ref[1:2, :]
    az = reft_ref[2:3, :]

    dx = tx - ax
    dy = ty - ay
    dz = tz - az
    d = dx * dx + dy * dy + dz * dz

    ji = lax.broadcasted_iota(jnp.int32, (_N, _RB), 0)
    work = (lax.bitcast_convert_type(d, jnp.int32) & jnp.int32(-8192)) | (
        jnp.where(d < _BALL2, ji, ji | jnp.int32(4096)))
    row_g = i * _RB + lax.broadcasted_iota(jnp.int32, (1, _RB), 1)
    idx_rows = [row_g]
    msk_rows = [jnp.ones((1, _RB), jnp.float32)]
    work = jnp.where(ji == row_g, jnp.int32(2147483647), work)
    for it in range(_NN - 1):
        v = jnp.min(work, axis=0, keepdims=True)
        idx_rows.append(v & jnp.int32(4095))
        msk_rows.append(((v & jnp.int32(4096)) == 0).astype(jnp.float32))
        if it < _NN - 2:
            work = jnp.where(work == v, jnp.int32(2147483647), work)
    idx_ref[...] = jnp.concatenate(idx_rows, axis=0)
    msk_ref[...] = jnp.concatenate(msk_rows, axis=0)


def _knn_call(ref_points, ref_t):
    nsteps = _N // _RB
    steps_per_w = _QW // _RB

    def omap(b, i):
        s = b * nsteps + i
        return (s // steps_per_w, 0, s % steps_per_w)

    return pl.pallas_call(
        _knn_body,
        grid=(_B, nsteps),
        in_specs=[
            pl.BlockSpec((None, _N, _C), lambda b, i: (b, 0, 0)),
            pl.BlockSpec((None, _C, _RB), lambda b, i: (b, 0, i)),
        ],
        out_specs=[
            pl.BlockSpec((None, _NN, _RB), omap),
            pl.BlockSpec((None, _NN, _RB), omap),
        ],
        out_shape=[
            jax.ShapeDtypeStruct((_NW, _NN, _QW), jnp.int32),
            jax.ShapeDtypeStruct((_NW, _NN, _QW), jnp.float32),
        ],
    )(ref_points, ref_t)


def _moments_sc(coords, idx16, msk16):
    mesh = plsc.VectorSubcoreMesh(core_axis_name="c", subcore_axis_name="s")

    cp = pltpu.CompilerParams()
    if "needs_layout_passes" in pltpu.CompilerParams.__dataclass_fields__:
        cp = dataclasses.replace(cp, needs_layout_passes=False)

    @functools.partial(
        pl.kernel,
        mesh=mesh,
        compiler_params=cp,
        out_type=jax.ShapeDtypeStruct((_NW, _NMOM, _QW), jnp.float32),
        scratch_types=[
            pltpu.VMEM((_N,), jnp.float32),
            pltpu.VMEM((_N,), jnp.float32),
            pltpu.VMEM((_N,), jnp.float32),
            pltpu.VMEM((_N,), jnp.float32),
            pltpu.VMEM((_N,), jnp.float32),
            pltpu.VMEM((_N,), jnp.float32),
            pltpu.VMEM((_NN, _QW), jnp.int32),
            pltpu.VMEM((_NN, _QW), jnp.float32),
            pltpu.VMEM((_NMOM, _QW), jnp.float32),
            pltpu.SemaphoreType.DMA,
        ],
    )
    def sc_kernel(coords_hbm, idx_hbm, msk_hbm, out_hbm,
                  rxv, ryv, rzv, pxv, pyv, pzv, iv, mv, ov, sem):
        wid = lax.axis_index("s") * 2 + lax.axis_index("c")
        qbase = wid * _QW
        batch = qbase // _N
        pltpu.async_copy(coords_hbm.at[batch, 0], rxv, sem).wait()
        pltpu.async_copy(coords_hbm.at[batch, 1], ryv, sem).wait()
        pltpu.async_copy(coords_hbm.at[batch, 2], rzv, sem).wait()
        pltpu.async_copy(coords_hbm.at[batch, 3], pxv, sem).wait()
        pltpu.async_copy(coords_hbm.at[batch, 4], pyv, sem).wait()
        pltpu.async_copy(coords_hbm.at[batch, 5], pzv, sem).wait()
        pltpu.async_copy(idx_hbm.at[wid], iv, sem).wait()
        pltpu.async_copy(msk_hbm.at[wid], mv, sem).wait()

        qoff = qbase - batch * _N

        @pl.loop(0, _QW, step=_QBLK)
        def _(base):
            qx = rxv[pl.ds(qoff + base, _QBLK)]
            qy = ryv[pl.ds(qoff + base, _QBLK)]
            qz = rzv[pl.ds(qoff + base, _QBLK)]
            zero = jnp.zeros((_QBLK,), jnp.float32)
            acc = [zero] * _NMOM
            for k in range(_NN):
                nidx = iv[k, pl.ds(base, _QBLK)]
                mk = mv[k, pl.ds(base, _QBLK)]
                gx = plsc.load_gather(rxv, [nidx]) - qx
                gy = plsc.load_gather(ryv, [nidx]) - qy
                gz = plsc.load_gather(rzv, [nidx]) - qz
                hx = plsc.load_gather(pxv, [nidx])
                hy = plsc.load_gather(pyv, [nidx])
                hz = plsc.load_gather(pzv, [nidx])
                acc[0] = acc[0] + mk
                acc[1] = acc[1] + mk * gx
                acc[2] = acc[2] + mk * gy
                acc[3] = acc[3] + mk * gz
                acc[4] = acc[4] + mk * gx * gx
                acc[5] = acc[5] + mk * gy * gy
                acc[6] = acc[6] + mk * gz * gz
                acc[7] = acc[7] + mk * gx * gy
                acc[8] = acc[8] + mk * gx * gz
                acc[9] = acc[9] + mk * gy * gz
                acc[10] = acc[10] + hx
                acc[11] = acc[11] + hy
                acc[12] = acc[12] + hz
                acc[13] = acc[13] + hx * hx
                acc[14] = acc[14] + hy * hy
                acc[15] = acc[15] + hz * hz
                acc[16] = acc[16] + hx * hy
                acc[17] = acc[17] + hx * hz
                acc[18] = acc[18] + hy * hz
            for s in range(_NMOM):
                ov[s, pl.ds(base, _QBLK)] = acc[s]

        pltpu.async_copy(ov, out_hbm.at[wid], sem).wait()

    return sc_kernel(coords, idx16, msk16)


def _finish_body(mom_ref, refq_ref, out_ref):
    g = pl.program_id(0)

    nb = mom_ref[0:1, :]
    m1x = mom_ref[1:2, :]
    m1y = mom_ref[2:3, :]
    m1z = mom_ref[3:4, :]
    m2xx = mom_ref[4:5, :]
    m2yy = mom_ref[5:6, :]
    m2zz = mom_ref[6:7, :]
    m2xy = mom_ref[7:8, :]
    m2xz = mom_ref[8:9, :]
    m2yz = mom_ref[9:10, :]
    p1x = mom_ref[10:11, :]
    p1y = mom_ref[11:12, :]
    p1z = mom_ref[12:13, :]
    p2xx = mom_ref[13:14, :]
    p2yy = mom_ref[14:15, :]
    p2zz = mom_ref[15:16, :]
    p2xy = mom_ref[16:17, :]
    p2xz = mom_ref[17:18, :]
    p2yz = mom_ref[18:19, :]
    ax = refq_ref[0:1, :]
    ay = refq_ref[1:2, :]
    az = refq_ref[2:3, :]

    nbinv = 1.0 / nb
    csx = m1x * nbinv
    csy = m1y * nbinv
    csz = m1z * nbinv
    cx = csx + ax
    cy = csy + ay
    cz = csz + az
    w = _NN - nb
    gxx = m2xx - nb * csx * csx + w * cx * cx
    gyy = m2yy - nb * csy * csy + w * cy * cy
    gzz = m2zz - nb * csz * csz + w * cz * cz
    gxy = m2xy - nb * csx * csy + w * cx * cy
    gxz = m2xz - nb * csx * csz + w * cx * cz
    gyz = m2yz - nb * csy * csz + w * cy * cz
    cond_ref = _cond_from_gram(gxx, gyy, gzz, gxy, gxz, gyz)

    cpx = p1x * nbinv
    cpy = p1y * nbinv
    cpz = p1z * nbinv
    hxx = p2xx - 2.0 * cpx * p1x + _NN * cpx * cpx
    hyy = p2yy - 2.0 * cpy * p1y + _NN * cpy * cpy
    hzz = p2zz - 2.0 * cpz * p1z + _NN * cpz * cpz
    hxy = p2xy - cpx * p1y - cpy * p1x + _NN * cpx * cpy
    hxz = p2xz - cpx * p1z - cpz * p1x + _NN * cpx * cpz
    hyz = p2yz - cpy * p1z - cpz * p1y + _NN * cpy * cpz
    cond_p = _cond_from_gram(hxx, hyy, hzz, hxy, hxz, hyz)

    diff = cond_p - cond_ref
    partial = jnp.sum(diff * diff, axis=1, keepdims=True)

    @pl.when(g == 0)
    def _():
        out_ref[0:1, 0:1] = jnp.zeros((1, 1), jnp.float32)

    out_ref[0:1, 0:1] = out_ref[0:1, 0:1] + partial

    @pl.when(g == _NW - 1)
    def _():
        out_ref[0:1, 0:1] = out_ref[0:1, 0:1] * (1.0 / _BN)


def _finish_call(mom, refq):
    return pl.pallas_call(
        _finish_body,
        grid=(_NW,),
        in_specs=[
            pl.BlockSpec((None, _NMOM, _QW), lambda g: (g, 0, 0)),
            pl.BlockSpec((_C, _QW), lambda g: (0, g)),
        ],
        out_specs=pl.BlockSpec((1, 1), lambda g: (0, 0)),
        out_shape=jax.ShapeDtypeStruct((1, 1), jnp.float32),
    )(mom, refq)


def kernel(ref_points, points):
    ref_t = ref_points.transpose(0, 2, 1)
    pts_t = points.transpose(0, 2, 1)
    coords = jnp.concatenate([ref_t, pts_t], axis=1)
    idx16, msk16 = _knn_call(ref_points, ref_t)
    mom = _moments_sc(coords, idx16, msk16)
    refq = ref_t.transpose(1, 0, 2).reshape(_C, _BN)
    out = _finish_call(mom, refq)
    return out[0, 0]
